# baseline (speedup 1.0000x reference)
import functools

import jax
import jax.numpy as jnp
from jax import lax
from jax.experimental import pallas as pl
from jax.experimental.pallas import tpu as pltpu
from jax.experimental.pallas import tpu_sc as plsc

N = 3200000
NUM_SHOWERS = 512
NUM_EVENTS = 8
NC = 2
NS = 16
NW = NC * NS
C = N // NW
CH = 20000
NCHUNK = C // CH
NV = CH // 16
TB = NUM_SHOWERS * NUM_EVENTS
TR = TB // 16

_mesh = plsc.VectorSubcoreMesh(core_axis_name="c", subcore_axis_name="s")


def _worker(c, s):
    return s * NC + c


def _seg_of(ivec, rs_rows):
    one = jnp.ones((16,), jnp.int32)
    zero = jnp.zeros((16,), jnp.int32)
    seg = jnp.where(ivec >= rs_rows[0], one, zero)
    for j in range(1, NUM_EVENTS - 1):
        seg = seg + jnp.where(ivec >= rs_rows[j], one, zero)
    return seg


@functools.partial(
    pl.kernel,
    out_type=jax.ShapeDtypeStruct((NC, TB), jnp.float32),
    mesh=_mesh,
    compiler_params=pltpu.CompilerParams(needs_layout_passes=False, use_tc_tiling_on_sc=False),
    scratch_types=[
        pltpu.VMEM((CH,), jnp.int32),
        pltpu.VMEM((CH,), jnp.int32),
        pltpu.VMEM((CH,), jnp.float32),
        pltpu.VMEM((CH,), jnp.float32),
        pltpu.VMEM((CH,), jnp.float32),
        pltpu.VMEM((CH,), jnp.float32),
        pltpu.VMEM((TB,), jnp.float32),
        pltpu.VMEM((16,), jnp.int32),
        pltpu.VMEM((TB // NS,), jnp.float32),
        pltpu.VMEM((TB // NS,), jnp.float32),
        pltpu.VMEM_SHARED((NS, TB), jnp.float32),
        pltpu.SemaphoreType.DMA,
        pltpu.SemaphoreType.DMA,
        pltpu.SemaphoreType.DMA,
        pltpu.SemaphoreType.DMA,
        pltpu.SemaphoreType.DMA,
        pltpu.SemaphoreType.DMA,
    ],
)
def _scatter_pass(sid_h, pcf_h, nrg_h, rs_h, part_h,
                  sid0, sid1, pcf0, pcf1, nrg0, nrg1,
                  tbl, rs_v, stage, acc, shared,
                  ss0, ss1, sp0, sp1, sn0, sn1):
    c = lax.axis_index("c")
    s = lax.axis_index("s")
    base = pl.multiple_of(_worker(c, s) * C, CH)

    pltpu.sync_copy(rs_h, rs_v.at[pl.ds(0, NUM_EVENTS + 1)])

    zf = jnp.zeros((16,), jnp.float32)

    def _zero(i, carry):
        tbl[pl.ds(i * 16, 16)] = zf
        return carry

    lax.fori_loop(0, TR, _zero, 0)

    rs_all = rs_v[pl.ds(0, 16)]
    rs_rows = [jnp.full((16,), rs_all[j + 1], jnp.int32)
               for j in range(NUM_EVENTS - 1)]
    iota = lax.iota(jnp.int32, 16)

    bufs = [(sid0, pcf0, nrg0, ss0, sp0, sn0),
            (sid1, pcf1, nrg1, ss1, sp1, sn1)]

    def _start(ci):
        sb, pb, nb, s_s, s_p, s_n = bufs[ci % 2]
        off = pl.multiple_of(base + ci * CH, CH)
        return (pltpu.async_copy(sid_h.at[pl.ds(off, CH)], sb, s_s),
                pltpu.async_copy(pcf_h.at[pl.ds(off, CH)], pb, s_p),
                pltpu.async_copy(nrg_h.at[pl.ds(off, CH)], nb, s_n))

    pending = _start(0)
    for ci in range(NCHUNK):
        for h in pending:
            h.wait()
        if ci + 1 < NCHUNK:
            pending = _start(ci + 1)
        sb, pb, nb = bufs[ci % 2][:3]
        cbase = base + ci * CH
        seg_lo = _seg_of(jnp.full((16,), cbase, jnp.int32), rs_rows)
        seg_hi = _seg_of(jnp.full((16,), cbase + (CH - 1), jnp.int32), rs_rows)
        segbase = 1 + (seg_lo << 9)

        def _fast(carry):
            def _body(i, c2):
                off = i * 16
                sid = sb[pl.ds(off, 16)]
                pcf = pb[pl.ds(off, 16)]
                nrg = nb[pl.ds(off, 16)]
                idx = sid + segbase
                val = nrg * jnp.where(sid >= 0, pcf, zf)
                plsc.addupdate_scatter(tbl, [idx], val)
                return c2
            return lax.fori_loop(0, NV, _body, carry)

        def _slow(carry):
            def _body(i, c2):
                off = i * 16
                sid = sb[pl.ds(off, 16)]
                pcf = pb[pl.ds(off, 16)]
                nrg = nb[pl.ds(off, 16)]
                ivec = cbase + off + iota
                seg = _seg_of(ivec, rs_rows)
                idx = sid + 1 + (seg << 9)
                val = nrg * jnp.where(sid >= 0, pcf, zf)
                plsc.addupdate_scatter(tbl, [idx], val)
                return c2
            return lax.fori_loop(0, NV, _body, carry)

        lax.cond(jnp.max(seg_hi) == jnp.max(seg_lo), _fast, _slow, 0)

    pltpu.sync_copy(tbl, shared.at[s])
    plsc.subcore_barrier()
    blk = TB // NS
    rbase = s * blk
    pltpu.sync_copy(shared.at[0, pl.ds(rbase, blk)], acc)
    for t in range(1, NS):
        pltpu.sync_copy(shared.at[t, pl.ds(rbase, blk)], stage)
        for r in range(blk // 16):
            acc[pl.ds(r * 16, 16)] = acc[pl.ds(r * 16, 16)] + stage[pl.ds(r * 16, 16)]
    pltpu.sync_copy(acc, part_h.at[c, pl.ds(rbase, blk)])


@functools.partial(
    pl.kernel,
    out_type=jax.ShapeDtypeStruct((N,), jnp.float32),
    mesh=_mesh,
    compiler_params=pltpu.CompilerParams(needs_layout_passes=False, use_tc_tiling_on_sc=False),
    scratch_types=[
        pltpu.VMEM((CH,), jnp.int32),
        pltpu.VMEM((CH,), jnp.int32),
        pltpu.VMEM((CH,), jnp.float32),
        pltpu.VMEM((CH,), jnp.float32),
        pltpu.VMEM((TB,), jnp.float32),
        pltpu.VMEM((TB,), jnp.float32),
        pltpu.VMEM((16,), jnp.int32),
        pltpu.SemaphoreType.DMA,
        pltpu.SemaphoreType.DMA,
        pltpu.SemaphoreType.DMA,
        pltpu.SemaphoreType.DMA,
    ],
)
def _gather_pass(sid_h, rs_h, part_h, out_h,
                 sid0, sid1, outb0, outb1, tblA, tblB, rs_v,
                 ss0, ss1, so0, so1):
    c = lax.axis_index("c")
    s = lax.axis_index("s")
    base = pl.multiple_of(_worker(c, s) * C, CH)

    pltpu.sync_copy(rs_h, rs_v.at[pl.ds(0, NUM_EVENTS + 1)])
    pltpu.sync_copy(part_h.at[0], tblA)
    pltpu.sync_copy(part_h.at[1], tblB)

    def _combine(i, carry):
        tblA[pl.ds(i * 16, 16)] = tblA[pl.ds(i * 16, 16)] + tblB[pl.ds(i * 16, 16)]
        return carry

    lax.fori_loop(0, TR, _combine, 0)

    rs_all = rs_v[pl.ds(0, 16)]
    rs_rows = [jnp.full((16,), rs_all[j + 1], jnp.int32)
               for j in range(NUM_EVENTS - 1)]
    iota = lax.iota(jnp.int32, 16)

    ins = [(sid0, ss0), (sid1, ss1)]
    outs = [(outb0, so0), (outb1, so1)]

    def _start_in(ci):
        sb, s_s = ins[ci % 2]
        off = pl.multiple_of(base + ci * CH, CH)
        return pltpu.async_copy(sid_h.at[pl.ds(off, CH)], sb, s_s)

    def _start_out(ci):
        ob, s_o = outs[ci % 2]
        off = pl.multiple_of(base + ci * CH, CH)
        return pltpu.async_copy(ob, out_h.at[pl.ds(off, CH)], s_o)

    pend_in = _start_in(0)
    pend_out = {}
    for ci in range(NCHUNK):
        pend_in.wait()
        if ci + 1 < NCHUNK:
            pend_in = _start_in(ci + 1)
        if ci - 2 in pend_out:
            pend_out.pop(ci - 2).wait()
        sb = ins[ci % 2][0]
        ob = outs[ci % 2][0]
        cbase = base + ci * CH
        seg_lo = _seg_of(jnp.full((16,), cbase, jnp.int32), rs_rows)
        seg_hi = _seg_of(jnp.full((16,), cbase + (CH - 1), jnp.int32), rs_rows)
        segbase = 1 + (seg_lo << 9)

        def _fast(carry):
            def _body(i, c2):
                off = i * 16
                sid = sb[pl.ds(off, 16)]
                idx = sid + segbase
                ob[pl.ds(off, 16)] = plsc.load_gather(tblA, [idx])
                return c2
            return lax.fori_loop(0, NV, _body, carry)

        def _slow(carry):
            def _body(i, c2):
                off = i * 16
                sid = sb[pl.ds(off, 16)]
                ivec = cbase + off + iota
                seg = _seg_of(ivec, rs_rows)
                idx = sid + 1 + (seg << 9)
                ob[pl.ds(off, 16)] = plsc.load_gather(tblA, [idx])
                return c2
            return lax.fori_loop(0, NV, _body, carry)

        lax.cond(jnp.max(seg_hi) == jnp.max(seg_lo), _fast, _slow, 0)
        pend_out[ci] = _start_out(ci)

    for ci in sorted(pend_out):
        pend_out[ci].wait()


def kernel(pred_sid, pred_corr_factor, rechit_energy, row_splits):
    sid = pred_sid[:, 0]
    pcf = pred_corr_factor[:, 0]
    nrg = rechit_energy[:, 0]
    parts = _scatter_pass(sid, pcf, nrg, row_splits)
    out = _gather_pass(sid, row_splits, parts)
    return out[:, None]
